# baseline (speedup 1.0000x reference)
import functools

import jax
import jax.numpy as jnp
from jax import lax
from jax.experimental import pallas as pl
from jax.experimental.pallas import tpu as pltpu
from jax.experimental.pallas import tpu_sc as plsc

VOCAB = 100000
DIM = 64
BATCH = 1024

_NC = 2
_NS = 16
_NW = _NC * _NS
_BPW = BATCH // _NW


_DIM2 = 2 * DIM


def _gather_body(table_hbm, idx_hbm, out_hbm, idx_v, rows_v, sem):
    wid = lax.axis_index("s") * _NC + lax.axis_index("c")
    base = wid * _BPW
    pltpu.sync_copy(idx_hbm.at[pl.ds(base, _BPW)], idx_v)
    pltpu.async_copy(table_hbm.at[idx_v], rows_v, sem).wait()
    pltpu.sync_copy(rows_v, out_hbm.at[pl.ds(base, _BPW)])


def _sc_gather(table_pairs, idx):
    call = functools.partial(
        pl.kernel,
        mesh=plsc.VectorSubcoreMesh(core_axis_name="c", subcore_axis_name="s"),
        out_type=jax.ShapeDtypeStruct((BATCH, _DIM2), jnp.float32),
        scratch_types=[
            pltpu.VMEM((_BPW,), jnp.int32),
            pltpu.VMEM((_BPW, _DIM2), jnp.float32),
            pltpu.SemaphoreType.DMA,
        ],
    )(_gather_body)
    return call(table_pairs, idx)


_TV = 512


def _matmul_body(z2_ref, p_ref, w_ref, b_ref, out_ref):
    lo = z2_ref[:, :DIM]
    hi = z2_ref[:, DIM:]
    z = lo + p_ref[...] * (hi - lo)
    out_ref[...] = (
        lax.dot_general(
            z,
            w_ref[...],
            (((1,), (1,)), ((), ())),
            preferred_element_type=jnp.float32,
        )
        + b_ref[...]
    )


def _projection(z2, p, out_W, out_b2d):
    grid = pl.cdiv(VOCAB, _TV)
    return pl.pallas_call(
        _matmul_body,
        grid=(grid,),
        in_specs=[
            pl.BlockSpec((BATCH, _DIM2), lambda i: (0, 0)),
            pl.BlockSpec((BATCH, 1), lambda i: (0, 0)),
            pl.BlockSpec((_TV, DIM), lambda i: (i, 0)),
            pl.BlockSpec((1, _TV), lambda i: (0, i)),
        ],
        out_specs=pl.BlockSpec((BATCH, _TV), lambda i: (0, i)),
        out_shape=jax.ShapeDtypeStruct((BATCH, VOCAB), jnp.float32),
        compiler_params=pltpu.CompilerParams(
            dimension_semantics=("arbitrary",),
        ),
    )(z2, p, out_W, out_b2d)


def kernel(x, emb_table, out_W, out_b):
    xi = x.astype(jnp.int32)
    table_pairs = emb_table.reshape(VOCAB // 2, _DIM2)
    z2 = _sc_gather(table_pairs, xi >> 1)
    p = (xi & 1).astype(jnp.float32).reshape(BATCH, 1)
    return _projection(z2, p, out_W, out_b.reshape(1, VOCAB))


# speedup vs baseline: 1.0259x; 1.0259x over previous
import functools

import jax
import jax.numpy as jnp
from jax import lax
from jax.experimental import pallas as pl
from jax.experimental.pallas import tpu as pltpu
from jax.experimental.pallas import tpu_sc as plsc

VOCAB = 100000
DIM = 64
BATCH = 1024

_NC = 2
_NS = 16
_NW = _NC * _NS
_BPW = BATCH // _NW


_DIM2 = 2 * DIM


def _gather_body(table_hbm, idx_hbm, out_hbm, idx_v, rows_v, sem):
    wid = lax.axis_index("s") * _NC + lax.axis_index("c")
    base = wid * _BPW
    pltpu.sync_copy(idx_hbm.at[pl.ds(base, _BPW)], idx_v)
    pltpu.async_copy(table_hbm.at[idx_v], rows_v, sem).wait()
    pltpu.sync_copy(rows_v, out_hbm.at[pl.ds(base, _BPW)])


def _sc_gather(table_pairs, idx):
    call = functools.partial(
        pl.kernel,
        mesh=plsc.VectorSubcoreMesh(core_axis_name="c", subcore_axis_name="s"),
        out_type=jax.ShapeDtypeStruct((BATCH, _DIM2), jnp.float32),
        scratch_types=[
            pltpu.VMEM((_BPW,), jnp.int32),
            pltpu.VMEM((_BPW, _DIM2), jnp.float32),
            pltpu.SemaphoreType.DMA,
        ],
    )(_gather_body)
    return call(table_pairs, idx)


_TV = 512


def _matmul_body(z2_ref, p_ref, w_ref, b_ref, out_ref, z_scr):
    @pl.when(pl.program_id(0) == 0)
    def _select():
        lo = z2_ref[:, :DIM]
        hi = z2_ref[:, DIM:]
        z_scr[...] = lo + p_ref[...] * (hi - lo)

    out_ref[...] = (
        lax.dot_general(
            z_scr[...],
            w_ref[...],
            (((1,), (1,)), ((), ())),
            preferred_element_type=jnp.float32,
        )
        + b_ref[...]
    )


def _projection(z2, p, out_W, out_b2d):
    grid = pl.cdiv(VOCAB, _TV)
    return pl.pallas_call(
        _matmul_body,
        grid=(grid,),
        in_specs=[
            pl.BlockSpec((BATCH, _DIM2), lambda i: (0, 0)),
            pl.BlockSpec((BATCH, 1), lambda i: (0, 0)),
            pl.BlockSpec((_TV, DIM), lambda i: (i, 0)),
            pl.BlockSpec((1, _TV), lambda i: (0, i)),
        ],
        out_specs=pl.BlockSpec((BATCH, _TV), lambda i: (0, i)),
        out_shape=jax.ShapeDtypeStruct((BATCH, VOCAB), jnp.float32),
        scratch_shapes=[pltpu.VMEM((BATCH, DIM), jnp.float32)],
        compiler_params=pltpu.CompilerParams(
            dimension_semantics=("arbitrary",),
        ),
    )(z2, p, out_W, out_b2d)


def kernel(x, emb_table, out_W, out_b):
    xi = x.astype(jnp.int32)
    table_pairs = emb_table.reshape(VOCAB // 2, _DIM2)
    z2 = _sc_gather(table_pairs, xi >> 1)
    p = (xi & 1).astype(jnp.float32).reshape(BATCH, 1)
    return _projection(z2, p, out_W, out_b.reshape(1, VOCAB))


# speedup vs baseline: 1.1619x; 1.1326x over previous
import functools

import jax
import jax.numpy as jnp
from jax import lax
from jax.experimental import pallas as pl
from jax.experimental.pallas import tpu as pltpu
from jax.experimental.pallas import tpu_sc as plsc

VOCAB = 100000
DIM = 64
BATCH = 1024

_NC = 2
_NS = 16
_NW = _NC * _NS
_BPW = BATCH // _NW


_DIM2 = 2 * DIM


def _gather_body(table_hbm, idx_hbm, out_hbm, idx_v, rows_v, sem):
    wid = lax.axis_index("s") * _NC + lax.axis_index("c")
    base = wid * _BPW
    pltpu.sync_copy(idx_hbm.at[pl.ds(base, _BPW)], idx_v)
    pltpu.async_copy(table_hbm.at[idx_v], rows_v, sem).wait()
    pltpu.sync_copy(rows_v, out_hbm.at[pl.ds(base, _BPW)])


def _sc_gather(table_pairs, idx):
    call = functools.partial(
        pl.kernel,
        mesh=plsc.VectorSubcoreMesh(core_axis_name="c", subcore_axis_name="s"),
        out_type=jax.ShapeDtypeStruct((BATCH, _DIM2), jnp.float32),
        scratch_types=[
            pltpu.VMEM((_BPW,), jnp.int32),
            pltpu.VMEM((_BPW, _DIM2), jnp.float32),
            pltpu.SemaphoreType.DMA,
        ],
    )(_gather_body)
    return call(table_pairs, idx)


_TV = 2048


def _matmul_body(z2_ref, p_ref, w_ref, b_ref, out_ref, z_scr):
    @pl.when(pl.program_id(0) == 0)
    def _select():
        lo = z2_ref[:, :DIM]
        hi = z2_ref[:, DIM:]
        z_scr[...] = lo + p_ref[...] * (hi - lo)

    out_ref[...] = (
        lax.dot_general(
            z_scr[...],
            w_ref[...],
            (((1,), (1,)), ((), ())),
            preferred_element_type=jnp.float32,
        )
        + b_ref[...]
    )


def _projection(z2, p, out_W, out_b2d):
    grid = pl.cdiv(VOCAB, _TV)
    return pl.pallas_call(
        _matmul_body,
        grid=(grid,),
        in_specs=[
            pl.BlockSpec((BATCH, _DIM2), lambda i: (0, 0)),
            pl.BlockSpec((BATCH, 1), lambda i: (0, 0)),
            pl.BlockSpec((_TV, DIM), lambda i: (i, 0)),
            pl.BlockSpec((1, _TV), lambda i: (0, i)),
        ],
        out_specs=pl.BlockSpec((BATCH, _TV), lambda i: (0, i)),
        out_shape=jax.ShapeDtypeStruct((BATCH, VOCAB), jnp.float32),
        scratch_shapes=[pltpu.VMEM((BATCH, DIM), jnp.float32)],
        compiler_params=pltpu.CompilerParams(
            dimension_semantics=("parallel",),
        ),
    )(z2, p, out_W, out_b2d)


def kernel(x, emb_table, out_W, out_b):
    xi = x.astype(jnp.int32)
    table_pairs = emb_table.reshape(VOCAB // 2, _DIM2)
    z2 = _sc_gather(table_pairs, xi >> 1)
    p = (xi & 1).astype(jnp.float32).reshape(BATCH, 1)
    return _projection(z2, p, out_W, out_b.reshape(1, VOCAB))


# speedup vs baseline: 2.6165x; 2.2519x over previous
import functools

import jax
import jax.numpy as jnp
from jax import lax
from jax.experimental import pallas as pl
from jax.experimental.pallas import tpu as pltpu
from jax.experimental.pallas import tpu_sc as plsc

VOCAB = 100000
DIM = 64
BATCH = 1024

_NC = 2
_NS = 16
_NW = _NC * _NS
_BPW = BATCH // _NW


_DIM2 = 2 * DIM


def _gather_body(table_hbm, idx_hbm, out_hbm, idx_v, rows_v, sem):
    wid = lax.axis_index("s") * _NC + lax.axis_index("c")
    base = wid * _BPW
    pltpu.sync_copy(idx_hbm.at[pl.ds(base, _BPW)], idx_v)
    pltpu.async_copy(table_hbm.at[idx_v], rows_v, sem).wait()
    pltpu.sync_copy(rows_v, out_hbm.at[pl.ds(base, _BPW)])


def _sc_gather(table_pairs, idx):
    call = functools.partial(
        pl.kernel,
        mesh=plsc.VectorSubcoreMesh(core_axis_name="c", subcore_axis_name="s"),
        out_type=jax.ShapeDtypeStruct((BATCH, _DIM2), jnp.float32),
        scratch_types=[
            pltpu.VMEM((_BPW,), jnp.int32),
            pltpu.VMEM((_BPW, _DIM2), jnp.float32),
            pltpu.SemaphoreType.DMA,
        ],
    )(_gather_body)
    return call(table_pairs, idx)


_TV = 2048


def _matmul_body(z2_ref, p_ref, wt_ref, b_ref, out_ref, z_scr):
    i = pl.program_id(0)

    @pl.when(i == 0)
    def _select():
        lo = z2_ref[:, :DIM]
        hi = z2_ref[:, DIM:]
        z_scr[...] = lo + p_ref[...] * (hi - lo)

    out_ref[...] = (
        lax.dot_general(
            wt_ref[...],
            z_scr[...],
            (((0,), (1,)), ((), ())),
            preferred_element_type=jnp.float32,
        )
        + b_ref[...]
    )


def _projection(z2, p, out_Wt, out_bc):
    grid = pl.cdiv(VOCAB, _TV)
    return pl.pallas_call(
        _matmul_body,
        grid=(grid,),
        in_specs=[
            pl.BlockSpec((BATCH, _DIM2), lambda i: (0, 0)),
            pl.BlockSpec((BATCH, 1), lambda i: (0, 0)),
            pl.BlockSpec((DIM, _TV), lambda i: (0, i)),
            pl.BlockSpec((_TV, 1), lambda i: (i, 0)),
        ],
        out_specs=pl.BlockSpec((_TV, BATCH), lambda i: (i, 0)),
        out_shape=jax.ShapeDtypeStruct((VOCAB, BATCH), jnp.float32),
        scratch_shapes=[pltpu.VMEM((BATCH, DIM), jnp.float32)],
        compiler_params=pltpu.CompilerParams(
            dimension_semantics=("parallel",),
        ),
    )(z2, p, out_Wt, out_bc)


def kernel(x, emb_table, out_W, out_b):
    xi = x.astype(jnp.int32)
    table_pairs = emb_table.reshape(VOCAB // 2, _DIM2)
    z2 = _sc_gather(table_pairs, xi >> 1)
    p = (xi & 1).astype(jnp.float32).reshape(BATCH, 1)
    logits_t = _projection(z2, p, out_W.T, out_b.reshape(VOCAB, 1))
    return logits_t.T


# speedup vs baseline: 3.2715x; 1.2503x over previous
import functools

import jax
import jax.numpy as jnp
from jax import lax
from jax.experimental import pallas as pl
from jax.experimental.pallas import tpu as pltpu
from jax.experimental.pallas import tpu_sc as plsc

VOCAB = 100000
DIM = 64
BATCH = 1024

_NC = 2
_NS = 16
_NW = _NC * _NS
_BPW = BATCH // _NW


def _gather_body(table_hbm, idx_hbm, out_hbm, idx_v, rows_v, sem):
    wid = lax.axis_index("s") * _NC + lax.axis_index("c")
    base = wid * _BPW
    pltpu.sync_copy(idx_hbm.at[pl.ds(base, _BPW)], idx_v)
    pltpu.async_copy(table_hbm.at[idx_v], rows_v, sem).wait()
    pltpu.sync_copy(rows_v, out_hbm.at[pl.ds(base, _BPW)])


def _sc_gather(table, idx):
    call = functools.partial(
        pl.kernel,
        mesh=plsc.VectorSubcoreMesh(core_axis_name="c", subcore_axis_name="s"),
        out_type=jax.ShapeDtypeStruct((BATCH, DIM), jnp.float32),
        scratch_types=[
            pltpu.VMEM((_BPW,), jnp.int32),
            pltpu.VMEM((_BPW, DIM), jnp.float32),
            pltpu.SemaphoreType.DMA,
        ],
        compiler_params=pltpu.CompilerParams(use_tc_tiling_on_sc=False),
    )(_gather_body)
    return call(table, idx)


_TV = 2048


def _matmul_body(z_ref, wt_ref, b_ref, out_ref):
    out_ref[...] = (
        lax.dot_general(
            wt_ref[...],
            z_ref[...],
            (((0,), (1,)), ((), ())),
            preferred_element_type=jnp.float32,
        )
        + b_ref[...].reshape(_TV, 1)
    )


def _projection(z, out_Wt, out_b):
    grid = pl.cdiv(VOCAB, _TV)
    return pl.pallas_call(
        _matmul_body,
        grid=(grid,),
        in_specs=[
            pl.BlockSpec((BATCH, DIM), lambda i: (0, 0)),
            pl.BlockSpec((DIM, _TV), lambda i: (0, i)),
            pl.BlockSpec((_TV,), lambda i: (i,)),
        ],
        out_specs=pl.BlockSpec((_TV, BATCH), lambda i: (i, 0)),
        out_shape=jax.ShapeDtypeStruct((VOCAB, BATCH), jnp.float32),
        compiler_params=pltpu.CompilerParams(
            dimension_semantics=("parallel",),
        ),
    )(z, out_Wt, out_b)


def kernel(x, emb_table, out_W, out_b):
    xi = x.astype(jnp.int32)
    z = _sc_gather(emb_table, xi)
    logits_t = _projection(z, out_W.T, out_b)
    return logits_t.T
